# baseline (speedup 1.0000x reference)
import math

import jax
import jax.numpy as jnp
import numpy as np
from jax import lax
from jax.experimental import pallas as pl
from jax.experimental.pallas import tpu as pltpu
from jax.experimental.pallas import tpu_sc as plsc

_SIZE = 32000
_N = 2048
_PAD = 0
_S32 = float(np.float32(0.1 / (_SIZE - 2)))
_C_ROW = (_SIZE - 2) * _S32 * math.log(_S32) + 0.9 * math.log(0.9)
_COEF = 0.9 - _S32

_NC, _NS, _L = 2, 16, 16
_NW = _NC * _NS

_R_TC = 768
_N_SC = _N - _R_TC
_RRPW = _N_SC // _NW
_NBUF = 4

_UNR = 16


def _row_reduce(buf):
    def body(i, acc):
        b = i * (_L * _UNR)
        for k in range(_UNR):
            acc = acc + buf[pl.ds(b + k * _L, _L)]
        return acc

    return lax.fori_loop(0, _SIZE // (_L * _UNR), body,
                         jnp.zeros((_L,), jnp.float32))


def _sc_body(x2, tgt, out, tgtd_v, acc_v, bufs, sems):
    wid = lax.axis_index("s") * _NC + lax.axis_index("c")
    dbase = _R_TC + wid * _RRPW
    pend = [None] * _NBUF
    for j in range(_NBUF - 1):
        pend[j] = pltpu.async_copy(x2.at[dbase + j], bufs[j], sems[j])
    pltpu.sync_copy(tgt.at[pl.ds(dbase, _RRPW)], tgtd_v.at[pl.ds(0, _RRPW)])

    total = jnp.zeros((_L,), jnp.float32)
    t16 = None
    for j in range(_RRPW):
        b = j % _NBUF
        if j % _L == 0:
            t16 = tgtd_v[pl.ds(j, _L)]
        nxt = j + _NBUF - 1
        if nxt < _RRPW:
            pend[nxt % _NBUF] = pltpu.async_copy(
                x2.at[dbase + nxt], bufs[nxt % _NBUF], sems[nxt % _NBUF])
        pend[b].wait()
        tval = t16[j % _L]
        rowvec = _row_reduce(bufs[b])
        tbase = pl.multiple_of((tval // _L) * _L, _L)
        tslice = bufs[b][pl.ds(tbase, _L)]
        lanes = lax.iota(jnp.int32, _L)
        x016 = bufs[b][pl.ds(0, _L)]
        head = jnp.where(
            lanes == 0,
            jnp.float32(_C_ROW) + jnp.float32(_S32) * x016,
            jnp.float32(0.0),
        )
        xtv = jnp.where(lanes == tval % _L,
                        jnp.float32(_COEF) * tslice, jnp.float32(0.0))
        contrib = head - xtv - jnp.float32(_S32) * rowvec
        total = total + jnp.where(tval != _PAD, contrib,
                                  jnp.zeros((_L,), jnp.float32))

    acc_v[...] = total
    pltpu.sync_copy(acc_v, out.at[pl.ds(wid * _L, _L)])


def _sc_entry(x2, tgt, out, tgtd_v, acc_v, buf0, buf1, buf2, buf3,
              sem0, sem1, sem2, sem3):
    _sc_body(x2, tgt, out, tgtd_v, acc_v,
             (buf0, buf1, buf2, buf3), (sem0, sem1, sem2, sem3))


_TGTD_PAD = ((_RRPW + _L - 1) // _L) * _L

_sc_part = pl.kernel(
    _sc_entry,
    out_type=jax.ShapeDtypeStruct((_NW * _L,), jnp.float32),
    mesh=plsc.VectorSubcoreMesh(core_axis_name="c", subcore_axis_name="s"),
    scratch_types=[
        pltpu.VMEM((_TGTD_PAD,), jnp.int32),
        pltpu.VMEM((_L,), jnp.float32),
        pltpu.VMEM((_SIZE,), jnp.float32),
        pltpu.VMEM((_SIZE,), jnp.float32),
        pltpu.VMEM((_SIZE,), jnp.float32),
        pltpu.VMEM((_SIZE,), jnp.float32),
        pltpu.SemaphoreType.DMA,
        pltpu.SemaphoreType.DMA,
        pltpu.SemaphoreType.DMA,
        pltpu.SemaphoreType.DMA,
    ],
)

_RB = 128
_NI = _R_TC // _RB


def _tc_body(tgt_ref, x_ref, out_ref, acc_ref):
    i = pl.program_id(0)

    @pl.when(i == 0)
    def _init():
        acc_ref[0] = 0.0

    xb = x_ref[...]
    tgtb = tgt_ref[...]
    mask = (tgtb != _PAD).astype(jnp.float32)
    rowsum = jnp.sum(xb, axis=1, keepdims=True)
    cols = lax.broadcasted_iota(jnp.int32, (_RB, _SIZE), 1)
    xt = jnp.sum(jnp.where(cols == tgtb, xb, 0.0), axis=1, keepdims=True)
    contrib = mask * (
        jnp.float32(_C_ROW)
        - jnp.float32(_COEF) * xt
        + jnp.float32(_S32) * xb[:, 0:1]
        - jnp.float32(_S32) * rowsum
    )
    acc_ref[0] = acc_ref[0] + jnp.sum(contrib)

    @pl.when(i == _NI - 1)
    def _emit():
        out_ref[...] = jnp.reshape(acc_ref[0], (1, 1))


_tc_reduce = pl.pallas_call(
    _tc_body,
    grid=(_NI,),
    in_specs=[
        pl.BlockSpec((_RB, 1), lambda i: (i, 0)),
        pl.BlockSpec((_RB, _SIZE), lambda i: (i, 0)),
    ],
    out_specs=pl.BlockSpec((1, 1), lambda i: (0, 0)),
    out_shape=jax.ShapeDtypeStruct((1, 1), jnp.float32),
    scratch_shapes=[pltpu.SMEM((1,), jnp.float32)],
)


def kernel(x, target):
    tgt32 = target.astype(jnp.int32)
    scp = _sc_part(x, tgt32)
    tc = _tc_reduce(jnp.reshape(tgt32, (_N, 1)), x)
    return jnp.sum(scp) + tc[0, 0]


# speedup vs baseline: 1.0790x; 1.0790x over previous
import math

import jax
import jax.numpy as jnp
import numpy as np
from jax import lax
from jax.experimental import pallas as pl
from jax.experimental.pallas import tpu as pltpu
from jax.experimental.pallas import tpu_sc as plsc

_SIZE = 32000
_N = 2048
_PAD = 0
_S32 = float(np.float32(0.1 / (_SIZE - 2)))
_C_ROW = (_SIZE - 2) * _S32 * math.log(_S32) + 0.9 * math.log(0.9)
_COEF = 0.9 - _S32

_NC, _NS, _L = 2, 16, 16
_NW = _NC * _NS

_R_TC = 896
_N_SC = _N - _R_TC
_RRPW = _N_SC // _NW
_NBUF = 3

_UNR = 16


def _row_reduce(buf):
    def body(i, acc):
        b = i * (_L * _UNR)
        for k in range(_UNR):
            acc = acc + buf[pl.ds(b + k * _L, _L)]
        return acc

    return lax.fori_loop(0, _SIZE // (_L * _UNR), body,
                         jnp.zeros((_L,), jnp.float32))


def _sc_body(x2, tgt, out, tgtd_v, acc_v, bufs, sems):
    wid = lax.axis_index("s") * _NC + lax.axis_index("c")
    dbase = _R_TC + wid * _RRPW
    pend = [None] * _NBUF
    for j in range(_NBUF - 1):
        pend[j] = pltpu.async_copy(x2.at[dbase + j], bufs[j], sems[j])
    pltpu.sync_copy(tgt.at[wid], tgtd_v)

    total = jnp.zeros((_L,), jnp.float32)
    t16 = None
    for j in range(_RRPW):
        b = j % _NBUF
        if j % _L == 0:
            t16 = tgtd_v[pl.ds(j, _L)]
        nxt = j + _NBUF - 1
        if nxt < _RRPW:
            pend[nxt % _NBUF] = pltpu.async_copy(
                x2.at[dbase + nxt], bufs[nxt % _NBUF], sems[nxt % _NBUF])
        pend[b].wait()
        tval = t16[j % _L]
        rowvec = _row_reduce(bufs[b])
        tbase = pl.multiple_of((tval // _L) * _L, _L)
        tslice = bufs[b][pl.ds(tbase, _L)]
        lanes = lax.iota(jnp.int32, _L)
        x016 = bufs[b][pl.ds(0, _L)]
        head = jnp.where(
            lanes == 0,
            jnp.float32(_C_ROW) + jnp.float32(_S32) * x016,
            jnp.float32(0.0),
        )
        xtv = jnp.where(lanes == tval % _L,
                        jnp.float32(_COEF) * tslice, jnp.float32(0.0))
        contrib = head - xtv - jnp.float32(_S32) * rowvec
        total = total + jnp.where(tval != _PAD, contrib,
                                  jnp.zeros((_L,), jnp.float32))

    acc_v[...] = total
    pltpu.sync_copy(acc_v, out.at[pl.ds(wid * _L, _L)])


def _sc_entry(x2, tgt, out, tgtd_v, acc_v, buf0, buf1, buf2,
              sem0, sem1, sem2):
    _sc_body(x2, tgt, out, tgtd_v, acc_v,
             (buf0, buf1, buf2), (sem0, sem1, sem2))


_TGTD_PAD = ((_RRPW + _L - 1) // _L) * _L

_sc_part = pl.kernel(
    _sc_entry,
    out_type=jax.ShapeDtypeStruct((_NW * _L,), jnp.float32),
    mesh=plsc.VectorSubcoreMesh(core_axis_name="c", subcore_axis_name="s"),
    scratch_types=[
        pltpu.VMEM((_TGTD_PAD,), jnp.int32),
        pltpu.VMEM((_L,), jnp.float32),
        pltpu.VMEM((_SIZE,), jnp.float32),
        pltpu.VMEM((_SIZE,), jnp.float32),
        pltpu.VMEM((_SIZE,), jnp.float32),
        pltpu.SemaphoreType.DMA,
        pltpu.SemaphoreType.DMA,
        pltpu.SemaphoreType.DMA,
    ],
)

_RB = 128
_NI = _R_TC // _RB


def _tc_body(tgt_ref, x_ref, out_ref, acc_ref):
    i = pl.program_id(0)

    @pl.when(i == 0)
    def _init():
        acc_ref[0] = 0.0

    xb = x_ref[...]
    tgtb = tgt_ref[...]
    mask = (tgtb != _PAD).astype(jnp.float32)
    rowsum = jnp.sum(xb, axis=1, keepdims=True)
    cols = lax.broadcasted_iota(jnp.int32, (_RB, _SIZE), 1)
    xt = jnp.sum(jnp.where(cols == tgtb, xb, 0.0), axis=1, keepdims=True)
    contrib = mask * (
        jnp.float32(_C_ROW)
        - jnp.float32(_COEF) * xt
        + jnp.float32(_S32) * xb[:, 0:1]
        - jnp.float32(_S32) * rowsum
    )
    acc_ref[0] = acc_ref[0] + jnp.sum(contrib)

    @pl.when(i == _NI - 1)
    def _emit():
        out_ref[...] = jnp.reshape(acc_ref[0], (1, 1))


_tc_reduce = pl.pallas_call(
    _tc_body,
    grid=(_NI,),
    in_specs=[
        pl.BlockSpec((_RB, 1), lambda i: (i, 0)),
        pl.BlockSpec((_RB, _SIZE), lambda i: (i, 0)),
    ],
    out_specs=pl.BlockSpec((1, 1), lambda i: (0, 0)),
    out_shape=jax.ShapeDtypeStruct((1, 1), jnp.float32),
    scratch_shapes=[pltpu.SMEM((1,), jnp.float32)],
)


def kernel(x, target):
    tgt32 = target.astype(jnp.int32)
    tgt_sc = jnp.pad(jnp.reshape(tgt32[_R_TC:], (_NW, _RRPW)),
                     ((0, 0), (0, _TGTD_PAD - _RRPW)))
    scp = _sc_part(x, tgt_sc)
    tc = _tc_reduce(jnp.reshape(tgt32, (_N, 1)), x)
    return jnp.sum(scp) + tc[0, 0]
